# baseline (speedup 1.0000x reference)
import functools

import jax
import jax.numpy as jnp
from jax import lax
from jax.experimental import pallas as pl
from jax.experimental.pallas import tpu as pltpu
from jax.experimental.pallas import tpu_sc as plsc

_B, _S, _D = 4, 8192, 768
_KEEP, _COMP = 144, 36
_T = 16
_TS = _S // _T
_L = _S - _KEEP
_NB = _S // 16
_CBR = 520
_OUTR = _KEEP + _COMP


def _tc_body(x_hbm, w1_ref, b1_ref, w2_ref,
             mask_ref, rank_ref, cb_ref,
             xs_ref, z_ref, bs_ref, sems):
    b = pl.program_id(0)
    t = pl.program_id(1)
    par = lax.rem(b, 2)

    def issue(batch, buf):
        for t2 in range(_T):
            pltpu.make_async_copy(
                x_hbm.at[batch, pl.ds(t2 * _TS, _TS), :],
                xs_ref.at[buf, pl.ds(t2 * _TS, _TS), :],
                sems.at[t2]).start()

    @pl.when((b == 0) & (t == 0))
    def _prime():
        issue(0, 0)

    @pl.when(t < _T)
    def _score():
        pltpu.make_async_copy(
            x_hbm.at[b, pl.ds(t * _TS, _TS), :],
            xs_ref.at[par, pl.ds(t * _TS, _TS), :],
            sems.at[t]).wait()
        xt = xs_ref[par, pl.ds(t * _TS, _TS), :]
        h = lax.dot_general(xt, w1_ref[...], (((1,), (1,)), ((), ())),
                            preferred_element_type=jnp.float32)
        h = jnp.maximum(h + b1_ref[0][None, :], 0.0)
        zt = lax.dot_general(w2_ref[...], h, (((1,), (1,)), ((), ())),
                             preferred_element_type=jnp.float32)
        z_ref[pl.ds(t, 1), :] = zt
        bs = jnp.sum(xt.reshape(32, 16, _D), axis=1)
        bs_ref[pl.ds(32 * t, 32), :] = bs

    @pl.when(t == _T)
    def _select():
        @pl.when(b + 1 < _B)
        def _prefetch_next():
            issue(b + 1, 1 - par)

        z = z_ref[...]
        bits = lax.bitcast_convert_type(z, jnp.int32)
        key = jnp.where(bits < 0, bits ^ jnp.int32(0x7FFFFFFF), bits)

        c0 = jnp.sum(jnp.where(key >= 0, 1, 0))
        thr0 = jnp.where(c0 >= _KEEP, jnp.int32(0), jnp.int32(-2147483648))

        def bit_body(i, thr):
            cand = thr | lax.shift_left(jnp.int32(1), 30 - i)
            cnt = jnp.sum(jnp.where(key >= cand, 1, 0))
            return jnp.where(cnt >= _KEEP, cand, thr)

        thr = lax.fori_loop(0, 31, bit_body, thr0)

        gt = key > thr
        eq = key == thr
        need = (_KEEP - jnp.sum(jnp.where(gt, 1, 0))).astype(jnp.float32)

        rr = lax.broadcasted_iota(jnp.int32, (_TS, _TS), 0)
        cc = lax.broadcasted_iota(jnp.int32, (_TS, _TS), 1)
        tri = (rr <= cc).astype(jnp.float32)
        r16 = lax.broadcasted_iota(jnp.int32, (_T, _T), 0)
        c16 = lax.broadcasted_iota(jnp.int32, (_T, _T), 1)
        stri = (c16 < r16).astype(jnp.float32)

        def flat_cumsum(m):
            cum = lax.dot_general(m, tri, (((1,), (0,)), ((), ())),
                                  preferred_element_type=jnp.float32)
            rows = jnp.sum(m, axis=1, keepdims=True)
            off = lax.dot_general(stri, rows, (((1,), (0,)), ((), ())),
                                  preferred_element_type=jnp.float32)
            return cum + off

        eqrank = flat_cumsum(eq.astype(jnp.float32))
        mask = gt | (eq & (eqrank <= need))
        maskf = mask.astype(jnp.float32)
        rank = flat_cumsum(maskf)
        mask_ref[0] = maskf
        rank_ref[0] = rank

        rq = lax.broadcasted_iota(jnp.int32, (_CBR, _NB), 0)
        cq = lax.broadcasted_iota(jnp.int32, (_CBR, _NB), 1)
        cbw = (cq < rq).astype(jnp.float32)
        cb = lax.dot_general(cbw, bs_ref[...], (((1,), (0,)), ((), ())),
                             preferred_element_type=jnp.float32)
        cb_ref[0] = cb


def _tc_call(x, W1, b1r, W2):
    grid = (_B, _T + 1)
    return pl.pallas_call(
        _tc_body,
        grid=grid,
        in_specs=[
            pl.BlockSpec(memory_space=pl.ANY),
            pl.BlockSpec((_D // 4, _D), lambda b, t: (0, 0)),
            pl.BlockSpec((1, _D // 4), lambda b, t: (0, 0)),
            pl.BlockSpec((1, _D // 4), lambda b, t: (0, 0)),
        ],
        out_specs=[
            pl.BlockSpec((1, _T, _TS), lambda b, t: (b, 0, 0)),
            pl.BlockSpec((1, _T, _TS), lambda b, t: (b, 0, 0)),
            pl.BlockSpec((1, _CBR, _D), lambda b, t: (b, 0, 0)),
        ],
        out_shape=[
            jax.ShapeDtypeStruct((_B, _T, _TS), jnp.float32),
            jax.ShapeDtypeStruct((_B, _T, _TS), jnp.float32),
            jax.ShapeDtypeStruct((_B, _CBR, _D), jnp.float32),
        ],
        scratch_shapes=[
            pltpu.VMEM((2, _S, _D), jnp.float32),
            pltpu.VMEM((_T, _TS), jnp.float32),
            pltpu.VMEM((_NB, _D), jnp.float32),
            pltpu.SemaphoreType.DMA((_T,)),
        ],
        compiler_params=pltpu.CompilerParams(
            vmem_limit_bytes=120 * 1024 * 1024),
    )(x, W1, b1r, W2)


def _sc_body(x2, rankh, maskh, cbh, out1,
             rank_v, mask_v, rejmap_v, kept_v, rows_v,
             corr_a, corr_b, ps_a, ps_b, pe_a, pe_b,
             cbe_a, cbe_b, cbs_a, cbs_b, acc_v,
             sem, sem2, semA, semB, semC, semD):
    core = lax.axis_index("c")
    sub = lax.axis_index("s")
    batch = core * 2 + sub // 8
    jj = lax.rem(sub, 8)
    base = batch * _S
    iota16 = lax.broadcasted_iota(jnp.int32, (16,), 0)

    cp1 = pltpu.async_copy(rankh.at[pl.ds(base, _S)], rank_v, sem)
    cp2 = pltpu.async_copy(maskh.at[pl.ds(base, _S)], mask_v, sem)
    cp1.wait()
    cp2.wait()

    def scan_body(c2, carry):
        for u in range(2):
            c = c2 * 2 + u
            rk = rank_v[pl.ds(c * 16, 16)].astype(jnp.int32)
            mk = mask_v[pl.ds(c * 16, 16)]
            keptm = mk > 0.5
            pos = iota16 + c * 16
            plsc.store_scatter(kept_v, [jnp.maximum(rk - 1, 0)], pos + base,
                               mask=keptm)
            plsc.store_scatter(rejmap_v, [pos - rk], pos,
                               mask=jnp.logical_not(keptm))
        return carry

    lax.fori_loop(0, _S // 32, scan_body, 0)

    @pl.when(jj < 6)
    def _kept():
        pltpu.async_copy(x2.at[kept_v.at[pl.ds(24 * jj, 24)]],
                         rows_v, sem).wait()
        for r in range(24):
            pltpu.async_copy(
                rows_v.at[r],
                out1.at[pl.ds((batch * _OUTR + 24 * jj + r) * _D, _D)],
                sem2).start()

    nseg = jnp.where(jj < 4, 5, 4)
    i_lo = jnp.where(jj < 4, 5 * jj, 4 * jj + 4)

    def splat(v):
        return jnp.full((16,), v, jnp.int32)

    def seg_scal(k):
        i = jnp.minimum(i_lo + k, _COMP - 1)
        s_i = (i * _L) // _COMP
        e_i = ((i + 1) * _L + _COMP - 1) // _COMP
        a_s = jnp.max(plsc.load_gather(rejmap_v, [splat(s_i)]))
        a_e_raw = jnp.max(plsc.load_gather(
            rejmap_v, [splat(jnp.minimum(e_i, _L - 1))]))
        a_e = jnp.where(e_i >= _L, _S, a_e_raw)
        m_lo = jnp.max(plsc.load_gather(
            rank_v, [splat(a_s)])).astype(jnp.int32)
        m_hi = jnp.where(
            a_e >= _S, _KEEP,
            jnp.max(plsc.load_gather(
                rank_v, [splat(jnp.minimum(a_e, _S - 1))])).astype(jnp.int32))
        return (i, s_i, e_i, a_s, a_e, m_lo, m_hi)

    scals = [seg_scal(k) for k in range(5)]
    bufs = [(cbe_a, cbs_a, ps_a, pe_a, corr_a, semA, semC),
            (cbe_b, cbs_b, ps_b, pe_b, corr_b, semB, semD)]

    def seg_copies(k):
        (i, s_i, e_i, a_s, a_e, m_lo, m_hi) = scals[k]
        cbe_v, cbs_v, ps_v, pe_v, corr_v, semX, semY = bufs[k % 2]
        qs = a_s // 16
        qe = a_e // 16
        ps_start = base + qs * 16
        pe_start = base + jnp.minimum(qe * 16, _S - 16)
        idxs = plsc.load_gather(
            kept_v, [jnp.minimum(m_lo + iota16, _KEEP - 1)])
        return [
            pltpu.async_copy(
                cbh.at[pl.ds((batch * _CBR + qe) * _D, _D)], cbe_v, semX),
            pltpu.async_copy(
                cbh.at[pl.ds((batch * _CBR + qs) * _D, _D)], cbs_v, semX),
            pltpu.async_copy(x2.at[pl.ds(ps_start, 16), :], ps_v, semX),
            pltpu.async_copy(x2.at[pl.ds(pe_start, 16), :], pe_v, semX),
            pltpu.async_copy(x2.at[idxs], corr_v, semY),
        ]

    @pl.when(0 < nseg)
    def _fire0():
        for cp in seg_copies(0):
            cp.start()

    for k in range(5):
        if k + 1 < 5:
            @pl.when(k + 1 < nseg)
            def _fire_next(k=k):
                for cp in seg_copies(k + 1):
                    cp.start()

        @pl.when(k < nseg)
        def _compute(k=k):
            (i, s_i, e_i, a_s, a_e, m_lo, m_hi) = scals[k]
            cbe_v, cbs_v, ps_v, pe_v, corr_v, semX, semY = bufs[k % 2]
            rs = lax.rem(a_s, 16)
            re = lax.rem(a_e, 16)
            cps = seg_copies(k)
            for cp in cps[:4]:
                cp.wait()

            acc = tuple(cbe_v[pl.ds(16 * c, 16)] - cbs_v[pl.ds(16 * c, 16)]
                        for c in range(48))

            def pe_body(r, a):
                return tuple(a[c] + pe_v[r, pl.ds(16 * c, 16)]
                             for c in range(48))

            acc = lax.fori_loop(0, re, pe_body, acc)

            def ps_body(r, a):
                return tuple(a[c] - ps_v[r, pl.ds(16 * c, 16)]
                             for c in range(48))

            acc = lax.fori_loop(0, rs, ps_body, acc)

            cps[4].wait()

            def cr_body(r, aa):
                return tuple(aa[c] - corr_v[r, pl.ds(16 * c, 16)]
                             for c in range(48))

            acc = lax.fori_loop(0, jnp.minimum(m_hi - m_lo, 16),
                                cr_body, acc)

            def ch_body(ch, a):
                idxs2 = plsc.load_gather(
                    kept_v,
                    [jnp.minimum(m_lo + ch * 16 + iota16, _KEEP - 1)])
                pltpu.async_copy(x2.at[idxs2], corr_v, semY).wait()
                nrows = jnp.minimum(m_hi - m_lo - ch * 16, 16)
                return lax.fori_loop(0, nrows, cr_body, a)

            nch = (m_hi - m_lo + 15) // 16
            acc = lax.fori_loop(1, nch, ch_body, acc)

            inv = jnp.where(e_i - s_i == 225,
                            jnp.float32(1.0 / 225.0), jnp.float32(1.0 / 224.0))
            for c in range(48):
                acc_v[k, pl.ds(16 * c, 16)] = acc[c] * inv
            pltpu.async_copy(
                acc_v.at[k],
                out1.at[pl.ds((batch * _OUTR + _KEEP + i) * _D, _D)],
                sem2).start()

    @pl.when(jj < 6)
    def _drain_kept():
        for r in range(24):
            pltpu.async_copy(
                rows_v.at[r],
                out1.at[pl.ds((batch * _OUTR + 24 * jj + r) * _D, _D)],
                sem2).wait()

    for k in range(5):
        @pl.when(k < nseg)
        def _drain_seg(k=k):
            (i, s_i, e_i, a_s, a_e, m_lo, m_hi) = scals[k]
            pltpu.async_copy(
                acc_v.at[k],
                out1.at[pl.ds((batch * _OUTR + _KEEP + i) * _D, _D)],
                sem2).wait()


_sc_mesh = plsc.VectorSubcoreMesh(core_axis_name="c", subcore_axis_name="s")

_sc_call = functools.partial(
    pl.kernel,
    out_type=jax.ShapeDtypeStruct((_B * _OUTR * _D,), jnp.float32),
    mesh=_sc_mesh,
    scratch_types=[
        pltpu.VMEM((_S,), jnp.float32),
        pltpu.VMEM((_S,), jnp.float32),
        pltpu.VMEM((_S,), jnp.int32),
        pltpu.VMEM((256,), jnp.int32),
        pltpu.VMEM((24, _D), jnp.float32),
        pltpu.VMEM((16, _D), jnp.float32),
        pltpu.VMEM((16, _D), jnp.float32),
        pltpu.VMEM((16, _D), jnp.float32),
        pltpu.VMEM((16, _D), jnp.float32),
        pltpu.VMEM((16, _D), jnp.float32),
        pltpu.VMEM((16, _D), jnp.float32),
        pltpu.VMEM((_D,), jnp.float32),
        pltpu.VMEM((_D,), jnp.float32),
        pltpu.VMEM((_D,), jnp.float32),
        pltpu.VMEM((_D,), jnp.float32),
        pltpu.VMEM((8, _D), jnp.float32),
        pltpu.SemaphoreType.DMA,
        pltpu.SemaphoreType.DMA,
        pltpu.SemaphoreType.DMA,
        pltpu.SemaphoreType.DMA,
        pltpu.SemaphoreType.DMA,
        pltpu.SemaphoreType.DMA,
    ],
    compiler_params=pltpu.CompilerParams(needs_layout_passes=False),
)(_sc_body)


@jax.jit
def kernel(x, W1, b1, W2, b2):
    del b2
    b1r = b1.reshape(1, _D // 4)
    mask, rank, cb = _tc_call(x, W1, b1r, W2)
    out1 = _sc_call(
        x.reshape(_B * _S, _D),
        rank.reshape(-1),
        mask.reshape(-1),
        cb.reshape(-1),
    )
    return out1.reshape(_B, _OUTR, _D)


# speedup vs baseline: 1.1381x; 1.1381x over previous
import functools

import jax
import jax.numpy as jnp
from jax import lax
from jax.experimental import pallas as pl
from jax.experimental.pallas import tpu as pltpu
from jax.experimental.pallas import tpu_sc as plsc

_B, _S, _D = 4, 8192, 768
_KEEP, _COMP = 144, 36
_T = 16
_TS = _S // _T
_L = _S - _KEEP
_NB = _S // 16
_CBR = 520
_OUTR = _KEEP + _COMP


def _tc_body(x_hbm, w1_ref, b1_ref, w2_ref,
             mask_ref, rank_ref, cb_ref,
             xs_ref, z_ref, bs_ref, sems):
    b = pl.program_id(0)
    t = pl.program_id(1)
    par = lax.rem(b, 2)

    def issue(batch, buf):
        for t2 in range(_T):
            pltpu.make_async_copy(
                x_hbm.at[batch, pl.ds(t2 * _TS, _TS), :],
                xs_ref.at[buf, pl.ds(t2 * _TS, _TS), :],
                sems.at[t2]).start()

    @pl.when((b == 0) & (t == 0))
    def _prime():
        issue(0, 0)

    @pl.when(t < _T)
    def _score():
        pltpu.make_async_copy(
            x_hbm.at[b, pl.ds(t * _TS, _TS), :],
            xs_ref.at[par, pl.ds(t * _TS, _TS), :],
            sems.at[t]).wait()
        xt = xs_ref[par, pl.ds(t * _TS, _TS), :]
        h = lax.dot_general(xt, w1_ref[...], (((1,), (1,)), ((), ())),
                            preferred_element_type=jnp.float32)
        h = jnp.maximum(h + b1_ref[0][None, :], 0.0)
        zt = lax.dot_general(w2_ref[...], h, (((1,), (1,)), ((), ())),
                             preferred_element_type=jnp.float32)
        z_ref[pl.ds(t, 1), :] = zt
        bs = jnp.sum(xt.reshape(32, 16, _D), axis=1)
        bs_ref[pl.ds(32 * t, 32), :] = bs

    @pl.when(t == _T)
    def _select():
        @pl.when(b + 1 < _B)
        def _prefetch_next():
            issue(b + 1, 1 - par)

        z = z_ref[...]
        bits = lax.bitcast_convert_type(z, jnp.int32)
        key = jnp.where(bits < 0, bits ^ jnp.int32(0x7FFFFFFF), bits)

        c0 = jnp.sum(jnp.where(key >= 0, 1, 0))
        thr0 = jnp.where(c0 >= _KEEP, jnp.int32(0), jnp.int32(-2147483648))

        def bit_body(i, thr):
            cand = thr | lax.shift_left(jnp.int32(1), 30 - i)
            cnt = jnp.sum(jnp.where(key >= cand, 1, 0))
            return jnp.where(cnt >= _KEEP, cand, thr)

        thr = lax.fori_loop(0, 31, bit_body, thr0)

        gt = key > thr
        eq = key == thr
        need = (_KEEP - jnp.sum(jnp.where(gt, 1, 0))).astype(jnp.float32)

        rr = lax.broadcasted_iota(jnp.int32, (_TS, _TS), 0)
        cc = lax.broadcasted_iota(jnp.int32, (_TS, _TS), 1)
        tri = (rr <= cc).astype(jnp.float32)
        r16 = lax.broadcasted_iota(jnp.int32, (_T, _T), 0)
        c16 = lax.broadcasted_iota(jnp.int32, (_T, _T), 1)
        stri = (c16 < r16).astype(jnp.float32)

        def flat_cumsum(m):
            cum = lax.dot_general(m, tri, (((1,), (0,)), ((), ())),
                                  preferred_element_type=jnp.float32)
            rows = jnp.sum(m, axis=1, keepdims=True)
            off = lax.dot_general(stri, rows, (((1,), (0,)), ((), ())),
                                  preferred_element_type=jnp.float32)
            return cum + off

        eqrank = flat_cumsum(eq.astype(jnp.float32))
        mask = gt | (eq & (eqrank <= need))
        maskf = mask.astype(jnp.float32)
        rank = flat_cumsum(maskf)
        mask_ref[0] = maskf
        rank_ref[0] = rank

        rq = lax.broadcasted_iota(jnp.int32, (_CBR, _NB), 0)
        cq = lax.broadcasted_iota(jnp.int32, (_CBR, _NB), 1)
        cbw = (cq < rq).astype(jnp.float32)
        cb = lax.dot_general(cbw, bs_ref[...], (((1,), (0,)), ((), ())),
                             preferred_element_type=jnp.float32)
        cb_ref[0] = cb


def _tc_call(x, W1, b1r, W2):
    grid = (_B, _T + 1)
    return pl.pallas_call(
        _tc_body,
        grid=grid,
        in_specs=[
            pl.BlockSpec(memory_space=pl.ANY),
            pl.BlockSpec((_D // 4, _D), lambda b, t: (0, 0)),
            pl.BlockSpec((1, _D // 4), lambda b, t: (0, 0)),
            pl.BlockSpec((1, _D // 4), lambda b, t: (0, 0)),
        ],
        out_specs=[
            pl.BlockSpec((1, _T, _TS), lambda b, t: (b, 0, 0)),
            pl.BlockSpec((1, _T, _TS), lambda b, t: (b, 0, 0)),
            pl.BlockSpec((1, _CBR, _D), lambda b, t: (b, 0, 0)),
        ],
        out_shape=[
            jax.ShapeDtypeStruct((_B, _T, _TS), jnp.float32),
            jax.ShapeDtypeStruct((_B, _T, _TS), jnp.float32),
            jax.ShapeDtypeStruct((_B, _CBR, _D), jnp.float32),
        ],
        scratch_shapes=[
            pltpu.VMEM((2, _S, _D), jnp.float32),
            pltpu.VMEM((_T, _TS), jnp.float32),
            pltpu.VMEM((_NB, _D), jnp.float32),
            pltpu.SemaphoreType.DMA((_T,)),
        ],
        compiler_params=pltpu.CompilerParams(
            vmem_limit_bytes=120 * 1024 * 1024),
    )(x, W1, b1r, W2)


def _sc_body(x2, rankh, maskh, cbh, out1,
             rank_v, mask_v, rejmap_v, kept_v, rows_v,
             corr_a, corr_b, ps_a, ps_b, pe_a, pe_b,
             cbe_a, cbe_b, cbs_a, cbs_b, acc_v,
             sem, sem2, semA, semB, semC, semD):
    core = lax.axis_index("c")
    sub = lax.axis_index("s")
    batch = core * 2 + sub // 8
    jj = lax.rem(sub, 8)
    base = batch * _S
    iota16 = lax.broadcasted_iota(jnp.int32, (16,), 0)

    cp1 = pltpu.async_copy(rankh.at[pl.ds(base, _S)], rank_v, sem)
    cp2 = pltpu.async_copy(maskh.at[pl.ds(base, _S)], mask_v, sem)
    cp1.wait()
    cp2.wait()

    def scan_body(c2, carry):
        for u in range(2):
            c = c2 * 2 + u
            rk = rank_v[pl.ds(c * 16, 16)].astype(jnp.int32)
            mk = mask_v[pl.ds(c * 16, 16)]
            keptm = mk > 0.5
            pos = iota16 + c * 16
            plsc.store_scatter(kept_v, [jnp.maximum(rk - 1, 0)], pos + base,
                               mask=keptm)
            plsc.store_scatter(rejmap_v, [pos - rk], pos,
                               mask=jnp.logical_not(keptm))
        return carry

    lax.fori_loop(0, _S // 32, scan_body, 0)

    @pl.when(jj < 6)
    def _kept():
        pltpu.async_copy(x2.at[kept_v.at[pl.ds(24 * jj, 24)]],
                         rows_v, sem).wait()
        for r in range(24):
            pltpu.make_async_copy(
                rows_v.at[r],
                out1.at[pl.ds((batch * _OUTR + 24 * jj + r) * _D, _D)],
                sem2).start()

    nseg = jnp.where(jj < 4, 5, 4)
    i_lo = jnp.where(jj < 4, 5 * jj, 4 * jj + 4)

    def splat(v):
        return jnp.full((16,), v, jnp.int32)

    def seg_scal(k):
        i = jnp.minimum(i_lo + k, _COMP - 1)
        s_i = (i * _L) // _COMP
        e_i = ((i + 1) * _L + _COMP - 1) // _COMP
        a_s = jnp.max(plsc.load_gather(rejmap_v, [splat(s_i)]))
        a_e_raw = jnp.max(plsc.load_gather(
            rejmap_v, [splat(jnp.minimum(e_i, _L - 1))]))
        a_e = jnp.where(e_i >= _L, _S, a_e_raw)
        m_lo = jnp.max(plsc.load_gather(
            rank_v, [splat(a_s)])).astype(jnp.int32)
        m_hi = jnp.where(
            a_e >= _S, _KEEP,
            jnp.max(plsc.load_gather(
                rank_v, [splat(jnp.minimum(a_e, _S - 1))])).astype(jnp.int32))
        return (i, s_i, e_i, a_s, a_e, m_lo, m_hi)

    scals = [seg_scal(k) for k in range(5)]
    bufs = [(cbe_a, cbs_a, ps_a, pe_a, corr_a, semA, semC),
            (cbe_b, cbs_b, ps_b, pe_b, corr_b, semB, semD)]

    def seg_copies(k):
        (i, s_i, e_i, a_s, a_e, m_lo, m_hi) = scals[k]
        cbe_v, cbs_v, ps_v, pe_v, corr_v, semX, semY = bufs[k % 2]
        qs = a_s // 16
        qe = a_e // 16
        ps_start = base + qs * 16
        pe_start = base + jnp.minimum(qe * 16, _S - 16)
        idxs = plsc.load_gather(
            kept_v, [jnp.minimum(m_lo + iota16, _KEEP - 1)])
        return [
            pltpu.make_async_copy(
                cbh.at[pl.ds((batch * _CBR + qe) * _D, _D)], cbe_v, semX),
            pltpu.make_async_copy(
                cbh.at[pl.ds((batch * _CBR + qs) * _D, _D)], cbs_v, semX),
            pltpu.make_async_copy(x2.at[pl.ds(ps_start, 16), :], ps_v, semX),
            pltpu.make_async_copy(x2.at[pl.ds(pe_start, 16), :], pe_v, semX),
            pltpu.make_async_copy(x2.at[idxs], corr_v, semY),
        ]

    @pl.when(0 < nseg)
    def _fire0():
        for cp in seg_copies(0):
            cp.start()

    for k in range(5):
        if k + 1 < 5:
            @pl.when(k + 1 < nseg)
            def _fire_next(k=k):
                for cp in seg_copies(k + 1):
                    cp.start()

        @pl.when(k < nseg)
        def _compute(k=k):
            (i, s_i, e_i, a_s, a_e, m_lo, m_hi) = scals[k]
            cbe_v, cbs_v, ps_v, pe_v, corr_v, semX, semY = bufs[k % 2]
            rs = lax.rem(a_s, 16)
            re = lax.rem(a_e, 16)
            cps = seg_copies(k)
            for cp in cps[:4]:
                cp.wait()

            acc = tuple(cbe_v[pl.ds(16 * c, 16)] - cbs_v[pl.ds(16 * c, 16)]
                        for c in range(48))

            def pe_body(r, a):
                return tuple(a[c] + pe_v[r, pl.ds(16 * c, 16)]
                             for c in range(48))

            acc = lax.fori_loop(0, re, pe_body, acc)

            def ps_body(r, a):
                return tuple(a[c] - ps_v[r, pl.ds(16 * c, 16)]
                             for c in range(48))

            acc = lax.fori_loop(0, rs, ps_body, acc)

            cps[4].wait()

            def cr_body(r, aa):
                return tuple(aa[c] - corr_v[r, pl.ds(16 * c, 16)]
                             for c in range(48))

            acc = lax.fori_loop(0, jnp.minimum(m_hi - m_lo, 16),
                                cr_body, acc)

            def ch_body(ch, a):
                idxs2 = plsc.load_gather(
                    kept_v,
                    [jnp.minimum(m_lo + ch * 16 + iota16, _KEEP - 1)])
                pltpu.async_copy(x2.at[idxs2], corr_v, semY).wait()
                nrows = jnp.minimum(m_hi - m_lo - ch * 16, 16)
                return lax.fori_loop(0, nrows, cr_body, a)

            nch = (m_hi - m_lo + 15) // 16
            acc = lax.fori_loop(1, nch, ch_body, acc)

            inv = jnp.where(e_i - s_i == 225,
                            jnp.float32(1.0 / 225.0), jnp.float32(1.0 / 224.0))
            for c in range(48):
                acc_v[k, pl.ds(16 * c, 16)] = acc[c] * inv
            pltpu.make_async_copy(
                acc_v.at[k],
                out1.at[pl.ds((batch * _OUTR + _KEEP + i) * _D, _D)],
                sem2).start()

    @pl.when(jj < 6)
    def _drain_kept():
        for r in range(24):
            pltpu.make_async_copy(
                rows_v.at[r],
                out1.at[pl.ds((batch * _OUTR + 24 * jj + r) * _D, _D)],
                sem2).wait()

    for k in range(5):
        @pl.when(k < nseg)
        def _drain_seg(k=k):
            (i, s_i, e_i, a_s, a_e, m_lo, m_hi) = scals[k]
            pltpu.make_async_copy(
                acc_v.at[k],
                out1.at[pl.ds((batch * _OUTR + _KEEP + i) * _D, _D)],
                sem2).wait()


_sc_mesh = plsc.VectorSubcoreMesh(core_axis_name="c", subcore_axis_name="s")

_sc_call = functools.partial(
    pl.kernel,
    out_type=jax.ShapeDtypeStruct((_B * _OUTR * _D,), jnp.float32),
    mesh=_sc_mesh,
    scratch_types=[
        pltpu.VMEM((_S,), jnp.float32),
        pltpu.VMEM((_S,), jnp.float32),
        pltpu.VMEM((_S,), jnp.int32),
        pltpu.VMEM((256,), jnp.int32),
        pltpu.VMEM((24, _D), jnp.float32),
        pltpu.VMEM((16, _D), jnp.float32),
        pltpu.VMEM((16, _D), jnp.float32),
        pltpu.VMEM((16, _D), jnp.float32),
        pltpu.VMEM((16, _D), jnp.float32),
        pltpu.VMEM((16, _D), jnp.float32),
        pltpu.VMEM((16, _D), jnp.float32),
        pltpu.VMEM((_D,), jnp.float32),
        pltpu.VMEM((_D,), jnp.float32),
        pltpu.VMEM((_D,), jnp.float32),
        pltpu.VMEM((_D,), jnp.float32),
        pltpu.VMEM((8, _D), jnp.float32),
        pltpu.SemaphoreType.DMA,
        pltpu.SemaphoreType.DMA,
        pltpu.SemaphoreType.DMA,
        pltpu.SemaphoreType.DMA,
        pltpu.SemaphoreType.DMA,
        pltpu.SemaphoreType.DMA,
    ],
    compiler_params=pltpu.CompilerParams(needs_layout_passes=False),
)(_sc_body)


@jax.jit
def kernel(x, W1, b1, W2, b2):
    del b2
    b1r = b1.reshape(1, _D // 4)
    mask, rank, cb = _tc_call(x, W1, b1r, W2)
    out1 = _sc_call(
        x.reshape(_B * _S, _D),
        rank.reshape(-1),
        mask.reshape(-1),
        cb.reshape(-1),
    )
    return out1.reshape(_B, _OUTR, _D)
